# Initial kernel scaffold; baseline (speedup 1.0000x reference)
import jax
import jax.numpy as jnp
from jax.experimental import pallas as pl


def kernel(support, query):
    raise NotImplementedError("write your pallas kernel here")


# speedup vs baseline: 3.3185x; 3.3185x over previous
import jax
import jax.numpy as jnp
from jax.experimental import pallas as pl

_TEMPERATURE = 64.0


def _protonet_body(s_ref, q_ref, o_ref):
    proto = jnp.sum(s_ref[...], axis=0) * (1.0 / s_ref.shape[0])
    q = q_ref[...]
    qn = jnp.sum(q * q, axis=1, keepdims=True)
    pn = jnp.sum(proto * proto, axis=1)[None, :]
    cross = jax.lax.dot_general(
        q, proto, (((1,), (1,)), ((), ())),
        preferred_element_type=jnp.float32,
        precision=jax.lax.Precision.HIGHEST,
    )
    o_ref[...] = (2.0 * cross - qn - pn) * (1.0 / _TEMPERATURE)


def kernel(support, query):
    n_batch, n_shot, n_way, emb_dim = support.shape
    n_query = query.shape[1] * n_way
    s = support.reshape(n_shot, n_way, emb_dim)
    q = query.reshape(n_batch * n_query, emb_dim)
    return pl.pallas_call(
        _protonet_body,
        out_shape=jax.ShapeDtypeStruct((n_batch * n_query, n_way), jnp.float32),
    )(s, q)
